# Initial kernel scaffold; baseline (speedup 1.0000x reference)
import jax
import jax.numpy as jnp
from jax.experimental import pallas as pl


def kernel(backbone_stem_w, backbone_stem_b, backbone_c1_w, backbone_c1_b, backbone_c2_w, backbone_c2_b, backbone_c3_w, backbone_c3_b, fpn_lat3_w, fpn_lat3_b, fpn_lat2_w, fpn_lat2_b, fpn_out3_w, fpn_out3_b, fpn_out2_w, fpn_out2_b, tower0_w, tower0_b, tower1_w, tower1_b, heads_w, heads_b, pool1, pool2, pool3, upmat, x):
    raise NotImplementedError("write your pallas kernel here")


# speedup vs baseline: 1.9350x; 1.9350x over previous
import jax
import jax.numpy as jnp
from jax.experimental import pallas as pl
from jax.experimental.pallas import tpu as pltpu

_IMGS_PER_STEP = 2


def _fused_body(x_ref, *args):
    (w_stem, b_stem, w_c1, b_c1, w_c2, b_c2, w_c3, b_c3,
     w_l3, b_l3, w_l2, b_l2, w_o3, b_o3, w_o2, b_o2,
     w_t0, b_t0, w_t1, b_t1, w_hd, b_hd,
     cls2_ref, cls3_ref, box2_ref, box3_ref, ctr2_ref, ctr3_ref,
     pad0, pad1, pad2, pad3, st0, st1, st2) = args

    f32, bf16 = jnp.float32, jnp.bfloat16
    B, C0, H0, W0 = x_ref.shape
    H1, W1 = H0 // 2, W0 // 2
    H2, W2 = H0 // 4, W0 // 4
    H3, W3 = H0 // 8, W0 // 8
    m2 = B * H2 * W2

    def zero_halo(pad, W):
        _, Hp, Wp, Cp = pad.shape
        zr = jnp.zeros((B, 1, Wp, Cp), pad.dtype)
        zc = jnp.zeros((B, Hp, 1, Cp), pad.dtype)
        pad[:, 0:1, :, :] = zr
        pad[:, Hp - 1:Hp, :, :] = zr
        pad[:, :, 7:8, :] = zc
        pad[:, :, 8 + W:9 + W, :] = zc

    zero_halo(pad0, W0)
    zero_halo(pad1, W1)
    zero_halo(pad2, W2)
    zero_halo(pad3, W3)

    def conv3x3(x4d, pad, w, b, relu):
        _, H, W, C = x4d.shape
        pad[:, 1:H + 1, 8:8 + W, :C] = x4d.astype(pad.dtype)
        rows = jnp.concatenate(
            [pad[:, :, 7 + kw:7 + kw + W, :C] for kw in range(3)],
            axis=-1)
        y = b[...].astype(f32)
        for kh in range(3):
            tap = rows[:, kh:kh + H].reshape(B * H * W, 3 * C)
            y = y + jnp.dot(tap, w[3 * C * kh:3 * C * (kh + 1), :],
                            preferred_element_type=f32)
        return jnp.maximum(y, 0.0) if relu else y

    def avgpool2(y2d, st, H, W, C):
        st[...] = y2d.astype(bf16).astype(f32).reshape(B, H, W, C)
        s = jnp.zeros((B, H // 2, W // 2, C), f32)
        for dh in range(2):
            for dw in range(2):
                s = s + st[:, pl.Slice(dh, H // 2, 2), pl.Slice(dw, W // 2, 2), :]
        return s * 0.25

    def conv1x1(a2d, w, b):
        return jnp.dot(a2d.astype(bf16), w[...],
                       preferred_element_type=f32) + b[...]

    x_nhwc = jnp.transpose(x_ref[...], (0, 2, 3, 1))
    c0 = conv3x3(x_nhwc, pad0, w_stem, b_stem, True)
    c1 = conv3x3(avgpool2(c0, st0, H0, W0, 16), pad1, w_c1, b_c1, True)
    c2 = conv3x3(avgpool2(c1, st1, H1, W1, 32), pad2, w_c2, b_c2, True)
    c3 = conv3x3(avgpool2(c2, st2, H2, W2, 48), pad3, w_c3, b_c3, True)

    F = w_l3.shape[1]
    lat3 = conv1x1(c3, w_l3, b_l3)
    lat2 = conv1x1(c2, w_l2, b_l2)
    u = lat3.astype(bf16).reshape(B, H3, W3, F)
    up3 = jnp.repeat(jnp.repeat(u, 2, axis=1), 2, axis=2)
    p2_in = lat2 + up3.astype(f32).reshape(B * H2 * W2, F)
    p3 = conv3x3(lat3.reshape(B, H3, W3, F), pad3, w_o3, b_o3, False)
    p2 = conv3x3(p2_in.reshape(B, H2, W2, F), pad2, w_o2, b_o2, False)

    def stacked_conv3x3(f2, f3, w, b, relu):
        C = f2.shape[-1]
        y2 = conv3x3(f2.reshape(B, H2, W2, C), pad2, w, b, relu)
        y3 = conv3x3(f3.reshape(B, H3, W3, C), pad3, w, b, relu)
        return y2, y3

    t2, t3 = stacked_conv3x3(p2, p3, w_t0, b_t0, True)
    t2, t3 = stacked_conv3x3(t2, t3, w_t1, b_t1, True)
    h2, h3 = stacked_conv3x3(t2, t3, w_hd, b_hd, False)

    ncls = cls2_ref.shape[1]
    z2 = jnp.transpose(h2.reshape(B, H2 * W2, -1), (0, 2, 1))
    z3 = jnp.transpose(h3.reshape(B, H3 * W3, -1), (0, 2, 1))
    z2 = z2.reshape(B, -1, H2, W2)
    z3 = z3.reshape(B, -1, H3, W3)
    cls2_ref[...] = z2[:, :ncls]
    box2_ref[...] = z2[:, ncls:ncls + 4]
    ctr2_ref[...] = z2[:, ncls + 4:ncls + 5]
    cls3_ref[...] = z3[:, :ncls]
    box3_ref[...] = z3[:, ncls:ncls + 4]
    ctr3_ref[...] = z3[:, ncls + 4:ncls + 5]


def _fused_pallas_call(params, x_nchw):
    N, C0, H0, W0 = x_nchw.shape
    H1, W1 = H0 // 2, W0 // 2
    H2, W2 = H0 // 4, W0 // 4
    H3, W3 = H0 // 8, W0 // 8

    B = _IMGS_PER_STEP
    while N % B:
        B //= 2

    c1_in = params[0].shape[1]
    c2_in = params[2].shape[1]
    c3_in = params[4].shape[1]
    two_f = params[18].shape[1]
    ncls = params[20].shape[1] - 5

    def _const(a):
        return pl.BlockSpec(a.shape, lambda n: (0, 0))

    def _out(ch, H, W):
        return (jax.ShapeDtypeStruct((N, ch, H, W), jnp.float32),
                pl.BlockSpec((B, ch, H, W), lambda n: (n, 0, 0, 0)))

    outs = [_out(ncls, H2, W2), _out(ncls, H3, W3),
            _out(4, H2, W2), _out(4, H3, W3),
            _out(1, H2, W2), _out(1, H3, W3)]

    return pl.pallas_call(
        _fused_body,
        out_shape=[o[0] for o in outs],
        grid=(N // B,),
        in_specs=[pl.BlockSpec((B, C0, H0, W0), lambda n: (n, 0, 0, 0))]
                 + [_const(a) for a in params],
        out_specs=[o[1] for o in outs],
        scratch_shapes=[
            pltpu.VMEM((B, H0 + 2, W0 + 16, C0), jnp.bfloat16),
            pltpu.VMEM((B, H1 + 2, W1 + 16, c1_in), jnp.bfloat16),
            pltpu.VMEM((B, H2 + 2, W2 + 16, max(c2_in, two_f)), jnp.bfloat16),
            pltpu.VMEM((B, H3 + 2, W3 + 16, max(c3_in, two_f)), jnp.bfloat16),
            pltpu.VMEM((B, H0, W0, c1_in), jnp.float32),
            pltpu.VMEM((B, H1, W1, c2_in), jnp.float32),
            pltpu.VMEM((B, H2, W2, c3_in), jnp.float32),
        ],
        compiler_params=pltpu.CompilerParams(
            dimension_semantics=("parallel",)),
    )(x_nchw, *params)


@jax.jit
def _forward(params, x_nchw):
    cls2, cls3, box2, box3, ctr2, ctr3 = _fused_pallas_call(params, x_nchw)
    return ([cls2, cls3], [box2, box3], [ctr2, ctr3], {2: 4, 3: 8})


def kernel(backbone_stem_w, backbone_stem_b,
           backbone_c1_w, backbone_c1_b,
           backbone_c2_w, backbone_c2_b,
           backbone_c3_w, backbone_c3_b,
           fpn_lat3_w, fpn_lat3_b,
           fpn_lat2_w, fpn_lat2_b,
           fpn_out3_w, fpn_out3_b,
           fpn_out2_w, fpn_out2_b,
           tower0_w, tower0_b,
           tower1_w, tower1_b,
           heads_w, heads_b,
           pool1, pool2, pool3, upmat, x):
    del pool1, pool2, pool3, upmat
    params = (backbone_stem_w, backbone_stem_b,
              backbone_c1_w, backbone_c1_b,
              backbone_c2_w, backbone_c2_b,
              backbone_c3_w, backbone_c3_b,
              fpn_lat3_w, fpn_lat3_b,
              fpn_lat2_w, fpn_lat2_b,
              fpn_out3_w, fpn_out3_b,
              fpn_out2_w, fpn_out2_b,
              tower0_w, tower0_b,
              tower1_w, tower1_b,
              heads_w, heads_b)
    return _forward(params, x)


# speedup vs baseline: 3.7646x; 1.9456x over previous
import jax
import jax.numpy as jnp
from jax.experimental import pallas as pl
from jax.experimental.pallas import tpu as pltpu

_IMGS_PER_STEP = 4
_HALO = 16


def _fused_body(x_ref, *args):
    (w_stem0, w_stem1, w_stem2, bt_stem,
     w_c1b0, w_c1b1, w_c1b2, bt_c1, ph0, pw0, ph1, pw1,
     w_c2, b_c2, w_c3, b_c3,
     w_l3, b_l3, w_l2, b_l2, w_o3, b_o3, w_o2, b_o2,
     w_t0, b_t0, w_t1, b_t1, w_hd, b_hd,
     cls2_ref, cls3_ref, box2_ref, box3_ref, ctr2_ref, ctr3_ref,
     a0, a1, pad2, pad3, st2) = args

    f32, bf16 = jnp.float32, jnp.bfloat16
    B, C0, H0, W0 = x_ref.shape
    H1, W1 = H0 // 2, W0 // 2
    H2, W2 = H0 // 4, W0 // 4
    H3, W3 = H0 // 8, W0 // 8
    HB = _HALO

    def banded_conv(a_scr, interior, H, wbands, bias):
        LK = interior.shape[-1]
        a_scr[:, HB - 1:HB, :] = jnp.zeros((B, 1, LK), f32)
        a_scr[:, HB + H:HB + H + 1, :] = jnp.zeros((B, 1, LK), f32)
        a_scr[:, HB:HB + H, :] = interior
        y = bias[...].astype(f32)
        for kh in range(3):
            tap = a_scr[:, HB - 1 + kh:HB - 1 + kh + H, :]
            y = y + jnp.dot(tap.reshape(B * H, LK).astype(bf16),
                            wbands[kh][...], preferred_element_type=f32)
        return jnp.maximum(y, 0.0)

    def packed_pool(y2d, ph, pw):
        y_r = y2d.astype(bf16).astype(f32)
        ws = jnp.dot(y_r, pw[...], preferred_element_type=f32)
        return jnp.dot(ph[...], ws, preferred_element_type=f32,
                       precision=jax.lax.Precision.HIGHEST)

    xp = jnp.concatenate([x_ref[:, c] for c in range(C0)], axis=-1)
    c0 = banded_conv(a0, xp, H0, (w_stem0, w_stem1, w_stem2), bt_stem)
    p1 = packed_pool(c0, ph0, pw0)
    c1 = banded_conv(a1, p1.reshape(B, H1, -1), H1,
                     (w_c1b0, w_c1b1, w_c1b2), bt_c1)
    p2in = packed_pool(c1, ph1, pw1)

    c2_in = w_c2.shape[0] // 9
    x2 = p2in.reshape(B, H2, W2, c2_in)

    def zero_halo(pad, W):
        _, Hp, Wp, Cp = pad.shape
        zr = jnp.zeros((B, 1, Wp, Cp), pad.dtype)
        zc = jnp.zeros((B, Hp, 1, Cp), pad.dtype)
        pad[:, 0:1, :, :] = zr
        pad[:, Hp - 1:Hp, :, :] = zr
        pad[:, :, 7:8, :] = zc
        pad[:, :, 8 + W:9 + W, :] = zc

    zero_halo(pad2, W2)
    zero_halo(pad3, W3)

    def conv3x3(x4d, pad, w, b, relu):
        _, H, W, C = x4d.shape
        pad[:, 1:H + 1, 8:8 + W, :C] = x4d.astype(pad.dtype)
        CP = 128 if C >= 48 else C
        z = jnp.zeros((B, H + 2, W, CP - C), pad.dtype) if CP > C else None
        pieces = []
        for kw in range(3):
            pieces.append(pad[:, :, 7 + kw:7 + kw + W, :C])
            if z is not None:
                pieces.append(z)
        rows = jnp.concatenate(pieces, axis=-1)
        y = b[...].astype(f32)
        for kh in range(3):
            tap = rows[:, kh:kh + H].reshape(B * H * W, 3 * CP)
            y = y + jnp.dot(tap, w[3 * CP * kh:3 * CP * (kh + 1), :],
                            preferred_element_type=f32)
        return jnp.maximum(y, 0.0) if relu else y

    def avgpool2(y2d, st, H, W, C):
        r = y2d.astype(bf16).astype(f32).reshape(B, H // 2, 2, W, C)
        st[...] = r[:, :, 0] + r[:, :, 1]
        s = (st[:, :, pl.Slice(0, W // 2, 2), :]
             + st[:, :, pl.Slice(1, W // 2, 2), :])
        return s * 0.25

    def conv1x1(a2d, w, b):
        return jnp.dot(a2d.astype(bf16), w[...],
                       preferred_element_type=f32) + b[...]

    c2 = conv3x3(x2, pad2, w_c2, b_c2, True)
    c3 = conv3x3(avgpool2(c2, st2, H2, W2, w_c2.shape[1]),
                 pad3, w_c3, b_c3, True)

    F = w_l3.shape[1]
    lat3 = conv1x1(c3, w_l3, b_l3)
    lat2 = conv1x1(c2, w_l2, b_l2)
    u = lat3.astype(bf16).reshape(B, H3, W3, F)
    up3 = jnp.repeat(jnp.repeat(u, 2, axis=1), 2, axis=2)
    p2_in = lat2 + up3.astype(f32).reshape(B * H2 * W2, F)
    p3 = conv3x3(lat3.reshape(B, H3, W3, F), pad3, w_o3, b_o3, False)
    p2 = conv3x3(p2_in.reshape(B, H2, W2, F), pad2, w_o2, b_o2, False)

    def stacked_conv3x3(f2, f3, w, b, relu):
        C = f2.shape[-1]
        y2 = conv3x3(f2.reshape(B, H2, W2, C), pad2, w, b, relu)
        y3 = conv3x3(f3.reshape(B, H3, W3, C), pad3, w, b, relu)
        return y2, y3

    t2, t3 = stacked_conv3x3(p2, p3, w_t0, b_t0, True)
    t2, t3 = stacked_conv3x3(t2, t3, w_t1, b_t1, True)
    h2, h3 = stacked_conv3x3(t2, t3, w_hd, b_hd, False)

    ncls = cls2_ref.shape[1]
    z2 = jnp.transpose(h2.reshape(B, H2 * W2, -1), (0, 2, 1))
    z3 = jnp.transpose(h3.reshape(B, H3 * W3, -1), (0, 2, 1))
    z2 = z2.reshape(B, -1, H2, W2)
    z3 = z3.reshape(B, -1, H3, W3)
    cls2_ref[...] = z2[:, :ncls]
    box2_ref[...] = z2[:, ncls:ncls + 4]
    ctr2_ref[...] = z2[:, ncls + 4:ncls + 5]
    cls3_ref[...] = z3[:, :ncls]
    box3_ref[...] = z3[:, ncls:ncls + 4]
    ctr3_ref[...] = z3[:, ncls + 4:ncls + 5]


def _band_weights(wmat, C, W, F, c_major):
    f32 = jnp.float32
    w4 = wmat.astype(f32).reshape(3, 3, C, F)
    v = jnp.arange(W)[:, None]
    wo = jnp.arange(W)[None, :]
    T = jnp.stack([(v == wo + kw - 1) for kw in range(3)]).astype(f32)
    pat = 'kvw,kcf->cvwf' if c_major else 'kvw,kcf->vcwf'
    return [jnp.einsum(pat, T, w4[kh]).reshape(W * C, W * F)
            .astype(jnp.bfloat16) for kh in range(3)]


def _pool_w_matrix(W, C):
    f32 = jnp.float32
    m = (jnp.arange(W)[:, None] // 2 == jnp.arange(W // 2)[None, :])
    return jnp.einsum('vw,cd->vcwd', m.astype(f32) * 0.5,
                      jnp.eye(C, dtype=f32)).reshape(W * C, (W // 2) * C)


def _respace_128(wmat):
    C = wmat.shape[0] // 9
    if C < 48:
        return wmat
    w4 = wmat.reshape(3, 3, C, -1)
    wp = jnp.zeros((3, 3, 128, w4.shape[-1]), wmat.dtype)
    wp = wp.at[:, :, :C, :].set(w4)
    return wp.reshape(9 * 128, -1)


def _pool_h_matrix(B, H):
    f32 = jnp.float32
    r = jnp.arange(B * H // 2)
    c = jnp.arange(B * H)
    return ((c[None, :] // 2 == r[:, None]).astype(f32) * 0.5)


def _fused_pallas_call(params, x_nchw):
    (w_stem, b_stem, w_c1, b_c1, w_c2, b_c2, w_c3, b_c3,
     w_l3, b_l3, w_l2, b_l2, w_o3, b_o3, w_o2, b_o2,
     w_t0, b_t0, w_t1, b_t1, w_hd, b_hd) = params
    N, C0, H0, W0 = x_nchw.shape
    H1, W1 = H0 // 2, W0 // 2
    H2, W2 = H0 // 4, W0 // 4
    H3, W3 = H0 // 8, W0 // 8

    B = _IMGS_PER_STEP
    while N % B:
        B //= 2

    c1_in = w_stem.shape[1]
    c2_in = w_c1.shape[1]
    two_f = w_t1.shape[1]
    ncls = w_hd.shape[1] - 5

    sb = _band_weights(w_stem, C0, W0, c1_in, c_major=True)
    cb = _band_weights(w_c1, c1_in, W1, c2_in, c_major=False)
    bt_stem = jnp.tile(b_stem, (1, W0))
    bt_c1 = jnp.tile(b_c1, (1, W1))
    pw0 = _pool_w_matrix(W0, c1_in)
    pw1 = _pool_w_matrix(W1, c2_in)
    ph0 = _pool_h_matrix(B, H0)
    ph1 = _pool_h_matrix(B, H1)

    kparams = (sb[0], sb[1], sb[2], bt_stem,
               cb[0], cb[1], cb[2], bt_c1, ph0, pw0, ph1, pw1,
               w_c2, b_c2, _respace_128(w_c3), b_c3,
               w_l3, b_l3, w_l2, b_l2, w_o3, b_o3, w_o2, b_o2,
               w_t0, b_t0, _respace_128(w_t1), b_t1,
               _respace_128(w_hd), b_hd)

    def _const(a):
        return pl.BlockSpec(a.shape, lambda n: (0, 0))

    def _out(ch, H, W):
        return (jax.ShapeDtypeStruct((N, ch, H, W), jnp.float32),
                pl.BlockSpec((B, ch, H, W), lambda n: (n, 0, 0, 0)))

    outs = [_out(ncls, H2, W2), _out(ncls, H3, W3),
            _out(4, H2, W2), _out(4, H3, W3),
            _out(1, H2, W2), _out(1, H3, W3)]

    HB = _HALO
    return pl.pallas_call(
        _fused_body,
        out_shape=[o[0] for o in outs],
        grid=(N // B,),
        in_specs=[pl.BlockSpec((B, C0, H0, W0), lambda n: (n, 0, 0, 0))]
                 + [_const(a) for a in kparams],
        out_specs=[o[1] for o in outs],
        scratch_shapes=[
            pltpu.VMEM((B, H0 + 2 * HB, W0 * C0), jnp.float32),
            pltpu.VMEM((B, H1 + 2 * HB, W1 * c1_in), jnp.float32),
            pltpu.VMEM((B, H2 + 2, W2 + 16, max(w_c1.shape[1], two_f)),
                       jnp.bfloat16),
            pltpu.VMEM((B, H3 + 2, W3 + 16, max(w_c2.shape[1], two_f)),
                       jnp.bfloat16),
            pltpu.VMEM((B, H2 // 2, W2, w_c2.shape[1]), jnp.float32),
        ],
        compiler_params=pltpu.CompilerParams(
            dimension_semantics=("parallel",)),
    )(x_nchw, *kparams)


@jax.jit
def _forward(params, x_nchw):
    cls2, cls3, box2, box3, ctr2, ctr3 = _fused_pallas_call(params, x_nchw)
    return ([cls2, cls3], [box2, box3], [ctr2, ctr3], {2: 4, 3: 8})


def kernel(backbone_stem_w, backbone_stem_b,
           backbone_c1_w, backbone_c1_b,
           backbone_c2_w, backbone_c2_b,
           backbone_c3_w, backbone_c3_b,
           fpn_lat3_w, fpn_lat3_b,
           fpn_lat2_w, fpn_lat2_b,
           fpn_out3_w, fpn_out3_b,
           fpn_out2_w, fpn_out2_b,
           tower0_w, tower0_b,
           tower1_w, tower1_b,
           heads_w, heads_b,
           pool1, pool2, pool3, upmat, x):
    del pool1, pool2, pool3, upmat
    params = (backbone_stem_w, backbone_stem_b,
              backbone_c1_w, backbone_c1_b,
              backbone_c2_w, backbone_c2_b,
              backbone_c3_w, backbone_c3_b,
              fpn_lat3_w, fpn_lat3_b,
              fpn_lat2_w, fpn_lat2_b,
              fpn_out3_w, fpn_out3_b,
              fpn_out2_w, fpn_out2_b,
              tower0_w, tower0_b,
              tower1_w, tower1_b,
              heads_w, heads_b)
    return _forward(params, x)
